# baseline (speedup 1.0000x reference)
import functools
import math

import jax
import jax.numpy as jnp
from jax.experimental import pallas as pl
from jax.experimental.pallas import tpu as pltpu

N = 100000
E = 1600000
F = 128
H = 64
C = 40
NITER = 10
CP = 48
RB = 2000
GRID = N // RB


def _mlp_body(x_ref, w0_ref, b0_ref, w1_ref, b1_ref, degp_ref,
              psa_ref, psb_ref, psc_ref, dinv_ref, rdinv_ref):
    xb = x_ref[...]
    h = jax.nn.relu(jnp.dot(xb, w0_ref[...], preferred_element_type=jnp.float32)
                    + b0_ref[...])
    preds = jnp.dot(h, w1_ref[...], preferred_element_type=jnp.float32) + b1_ref[...]
    deg = degp_ref[0, :, :] + degp_ref[1, :, :] + 1.0
    dinv = jax.lax.rsqrt(deg)
    dinv_ref[...] = dinv
    rdinv_ref[...] = jnp.sqrt(deg)
    ps = preds * dinv
    psa_ref[...] = ps[:, 0:16]
    psb_ref[...] = ps[:, 16:32]
    psc_ref[...] = jnp.pad(ps[:, 32:40], ((0, 0), (0, 8)))


def _mlp_prep(x, W0, b0, W1, b1, degp):
    return pl.pallas_call(
        _mlp_body,
        grid=(GRID,),
        in_specs=[
            pl.BlockSpec((RB, F), lambda i: (i, 0)),
            pl.BlockSpec((F, H), lambda i: (0, 0)),
            pl.BlockSpec((1, H), lambda i: (0, 0)),
            pl.BlockSpec((H, C), lambda i: (0, 0)),
            pl.BlockSpec((1, C), lambda i: (0, 0)),
            pl.BlockSpec((2, RB, 1), lambda i: (0, i, 0)),
        ],
        out_specs=[
            pl.BlockSpec((RB, 16), lambda i: (i, 0)),
            pl.BlockSpec((RB, 16), lambda i: (i, 0)),
            pl.BlockSpec((RB, 16), lambda i: (i, 0)),
            pl.BlockSpec((RB, 1), lambda i: (i, 0)),
            pl.BlockSpec((RB, 1), lambda i: (i, 0)),
        ],
        out_shape=[
            jax.ShapeDtypeStruct((N, 16), jnp.float32),
            jax.ShapeDtypeStruct((N, 16), jnp.float32),
            jax.ShapeDtypeStruct((N, 16), jnp.float32),
            jax.ShapeDtypeStruct((N, 1), jnp.float32),
            jax.ShapeDtypeStruct((N, 1), jnp.float32),
        ],
    )(x, W0, b0.reshape(1, H), W1, b1.reshape(1, C), degp)


def _halt_body(ra_ref, rb_ref, rc_ref, pa_ref, pb_ref, pc_ref,
               dinv_ref, rdinv_ref, steps_ref, sumh_ref, cont_ref, xacc_ref,
               wh_ref, bh_ref,
               oa_ref, ob_ref, oc_ref, osteps_ref, osumh_ref, ocont_ref,
               oxacc_ref):
    raw = jnp.concatenate([ra_ref[...], rb_ref[...], rc_ref[...]], axis=1)
    ps = jnp.concatenate([pa_ref[...], pb_ref[...], pc_ref[...]], axis=1)
    dinv = dinv_ref[...]
    prop = dinv * (raw + ps)
    old_prop = rdinv_ref[...] * ps
    hh = jax.nn.sigmoid(
        jnp.dot(prop, wh_ref[...], preferred_element_type=jnp.float32)
        + bh_ref[0, 0])
    steps = steps_ref[...]
    sum_h = sumh_ref[...]
    cont = cont_ref[...]
    prob = jnp.where((sum_h + hh < 0.99) & (cont > 0.0), 1.0, 0.0)
    steps = steps + prob
    sum_h = sum_h + prob * hh
    condition = prob * jnp.where(steps < float(NITER), 1.0, 0.0)
    p = jnp.where(condition > 0.0, sum_h, 1.0 - sum_h)
    oxacc_ref[...] = xacc_ref[...] + (
        prop[:, 0:C] * p + old_prop[:, 0:C] * (1.0 - p)) * cont
    nps = dinv * prop
    oa_ref[...] = nps[:, 0:16]
    ob_ref[...] = nps[:, 16:32]
    oc_ref[...] = nps[:, 32:48]
    osteps_ref[...] = steps
    osumh_ref[...] = sum_h
    ocont_ref[...] = cont * prob


def _halt(raw3, ps3, dinv, rdinv, steps, sum_h, cont, xacc, Wh48, bh):
    c16 = lambda: pl.BlockSpec((RB, 16), lambda i: (i, 0))
    c1 = lambda: pl.BlockSpec((RB, 1), lambda i: (i, 0))
    return pl.pallas_call(
        _halt_body,
        grid=(GRID,),
        in_specs=[c16(), c16(), c16(), c16(), c16(), c16(),
                  c1(), c1(), c1(), c1(), c1(),
                  pl.BlockSpec((RB, C), lambda i: (i, 0)),
                  pl.BlockSpec((CP, 1), lambda i: (0, 0)),
                  pl.BlockSpec((1, 1), lambda i: (0, 0))],
        out_specs=[c16(), c16(), c16(), c1(), c1(), c1(),
                   pl.BlockSpec((RB, C), lambda i: (i, 0))],
        out_shape=[
            jax.ShapeDtypeStruct((N, 16), jnp.float32),
            jax.ShapeDtypeStruct((N, 16), jnp.float32),
            jax.ShapeDtypeStruct((N, 16), jnp.float32),
            jax.ShapeDtypeStruct((N, 1), jnp.float32),
            jax.ShapeDtypeStruct((N, 1), jnp.float32),
            jax.ShapeDtypeStruct((N, 1), jnp.float32),
            jax.ShapeDtypeStruct((N, C), jnp.float32),
        ],
    )(raw3[0], raw3[1], raw3[2], ps3[0], ps3[1], ps3[2],
      dinv, rdinv, steps, sum_h, cont, xacc, Wh48, bh)


def _final_body(xacc_ref, steps_ref, sumh_ref, out_ref, osteps_ref, orem_ref):
    steps = steps_ref[...]
    o = xacc_ref[...] / steps
    m = jnp.max(o, axis=1, keepdims=True)
    z = o - m
    lse = jnp.log(jnp.sum(jnp.exp(z), axis=1, keepdims=True))
    out_ref[...] = z - lse
    osteps_ref[...] = steps
    orem_ref[...] = 1.0 - sumh_ref[...]


def _final(xacc, steps, sum_h):
    c1 = lambda: pl.BlockSpec((RB, 1), lambda i: (i, 0))
    return pl.pallas_call(
        _final_body,
        grid=(GRID,),
        in_specs=[pl.BlockSpec((RB, C), lambda i: (i, 0)), c1(), c1()],
        out_specs=[pl.BlockSpec((RB, C), lambda i: (i, 0)), c1(), c1()],
        out_shape=[
            jax.ShapeDtypeStruct((N, C), jnp.float32),
            jax.ShapeDtypeStruct((N, 1), jnp.float32),
            jax.ShapeDtypeStruct((N, 1), jnp.float32),
        ],
    )(xacc, steps, sum_h)


def _propagate(ps3, src, dst):
    raws = []
    for k in range(3):
        msgs = ps3[k][src]
        raws.append(jnp.zeros((N, 16), jnp.float32).at[dst].add(msgs))
    return raws


def kernel(x, edge_index, W0, b0, W1, b1, Wh, bh):
    src = edge_index[0]
    dst = edge_index[1]
    degp = jnp.zeros((2, N), jnp.float32).at[1, src].add(1.0)
    degp = degp.reshape(2, N, 1)

    psa, psb, psc, dinv, rdinv = _mlp_prep(x, W0, b0, W1, b1, degp)
    ps3 = [psa, psb, psc]

    Wh48 = jnp.pad(Wh, ((0, CP - C), (0, 0)))
    bh2 = bh.reshape(1, 1)
    steps = jnp.ones((N, 1), jnp.float32)
    sum_h = jnp.zeros((N, 1), jnp.float32)
    cont = jnp.ones((N, 1), jnp.float32)
    xacc = jnp.zeros((N, C), jnp.float32)

    for _ in range(NITER):
        raw3 = _propagate(ps3, src, dst)
        a, b, c, steps, sum_h, cont, xacc = _halt(
            raw3, ps3, dinv, rdinv, steps, sum_h, cont, xacc, Wh48, bh2)
        ps3 = [a, b, c]

    logits, steps_o, rem = _final(xacc, steps, sum_h)
    return logits, steps_o.reshape(N), rem.reshape(N)


# speedup vs baseline: 10.0403x; 10.0403x over previous
import functools
import math

import jax
import jax.numpy as jnp
from jax import lax
from jax.experimental import pallas as pl
from jax.experimental.pallas import tpu as pltpu
from jax.experimental.pallas import tpu_sc as plsc

N = 100000
E = 1600000
F = 128
H = 64
C = 40
NITER = 10
CH = 8
NCH = C // CH

NACC = 100352
STRIPE = NACC // 16
CO = STRIPE // 4

E_PAD = 1601536
GROUPS = E_PAD // 128
GM = 17
LMS = GM * 128
GPTD = GROUPS // 32
NBD = GPTD // GM

RB = STRIPE
GRID = NACC // RB

_mesh = plsc.VectorSubcoreMesh(core_axis_name="c", subcore_axis_name="s")


def _deg_body(srcg, zeros1, degp, onesb, idxb, acc1, obuf1, dsem):
    cid = lax.axis_index("c")
    sid = lax.axis_index("s")
    for k in range(128 // 16):
        onesb[pl.ds(k * 16, 16)] = jnp.full((16,), 1.0, jnp.float32)
    @pl.loop(0, 2)
    def _z(k):
        off = sid * STRIPE + k * (STRIPE // 2)
        pltpu.sync_copy(zeros1.at[pl.ds(off, STRIPE // 2)], obuf1)
        pltpu.sync_copy(obuf1, acc1.at[pl.ds(off, STRIPE // 2)])

    plsc.subcore_barrier()
    tb = cid * (GROUPS // 2) + sid * GPTD

    @pl.loop(0, NBD)
    def _blk(b):
        gb = tb + b * GM
        pltpu.sync_copy(srcg.at[pl.ds(gb, GM)], idxb)
        for g in range(GM):
            pltpu.async_copy(onesb, acc1.at[idxb.at[g]], dsem, add=True)
        for g in range(GM):
            pltpu.make_async_copy(onesb, acc1.at[idxb.at[g]], dsem).wait()

    plsc.subcore_barrier()

    @pl.loop(0, 2)
    def _co(k):
        off = sid * STRIPE + k * (STRIPE // 2)
        pltpu.sync_copy(acc1.at[pl.ds(off, STRIPE // 2)], obuf1)
        pltpu.sync_copy(obuf1, degp.at[cid, pl.ds(off, STRIPE // 2)])


@functools.partial(
    pl.kernel,
    out_type=jax.ShapeDtypeStruct((2, NACC), jnp.float32),
    mesh=_mesh,
    compiler_params=pltpu.CompilerParams(use_tc_tiling_on_sc=False),
    scratch_types=[
        pltpu.VMEM((128,), jnp.float32),
        pltpu.VMEM((GM, 128), jnp.int32),
        pltpu.VMEM_SHARED((NACC,), jnp.float32),
        pltpu.VMEM((STRIPE // 2,), jnp.float32),
        pltpu.SemaphoreType.DMA,
    ],
)
def _deg_sc(srcg, zeros1, degp, onesb, idxb, acc1, obuf1, dsem):
    _deg_body(srcg, zeros1, degp, onesb, idxb, acc1, obuf1, dsem)


@functools.partial(
    pl.kernel,
    out_type=jax.ShapeDtypeStruct((2, NACC, CH), jnp.float32),
    mesh=_mesh,
    compiler_params=pltpu.CompilerParams(use_tc_tiling_on_sc=False),
    scratch_types=[
        pltpu.VMEM((GM, 128), jnp.int32),
        pltpu.VMEM((GM, 128), jnp.int32),
        pltpu.VMEM((LMS, CH), jnp.float32),
        pltpu.VMEM_SHARED((NACC, CH), jnp.float32),
        pltpu.SemaphoreType.DMA,
    ],
)
def _prop_sc(ps, srcg, dstg, zeros2, rawp, srcb, dstb, rows, acc, sem):
    cid = lax.axis_index("c")
    sid = lax.axis_index("s")

    pltpu.sync_copy(zeros2, rows.at[pl.ds(0, CO)])

    @pl.loop(0, 4)
    def _z(k):
        off = sid * STRIPE + k * CO
        pltpu.sync_copy(rows.at[pl.ds(0, CO)], acc.at[pl.ds(off, CO)])

    plsc.subcore_barrier()
    tb = cid * (GROUPS // 2) + sid * GPTD

    @pl.loop(0, NBD)
    def _blk(b):
        gb = tb + b * GM
        pltpu.sync_copy(srcg.at[pl.ds(gb, GM)], srcb)
        pltpu.sync_copy(dstg.at[pl.ds(gb, GM)], dstb)
        for g in range(GM):
            pltpu.async_copy(ps.at[srcb.at[g]], rows.at[pl.ds(g * 128, 128)],
                             sem)
        for g in range(GM):
            pltpu.make_async_copy(ps.at[srcb.at[g]],
                                  rows.at[pl.ds(g * 128, 128)], sem).wait()
        for g in range(GM):
            pltpu.async_copy(rows.at[pl.ds(g * 128, 128)], acc.at[dstb.at[g]],
                             sem, add=True)
        for g in range(GM):
            pltpu.make_async_copy(rows.at[pl.ds(g * 128, 128)],
                                  acc.at[dstb.at[g]], sem).wait()

    plsc.subcore_barrier()

    @pl.loop(0, 4)
    def _co(k):
        off = sid * STRIPE + k * CO
        pltpu.sync_copy(acc.at[pl.ds(off, CO)], rows.at[pl.ds(0, CO)])
        pltpu.sync_copy(rows.at[pl.ds(0, CO)], rawp.at[cid, pl.ds(off, CO)])


def _mlp_body(xt_ref, w0t_ref, b0_ref, w1t_ref, b1_ref, degp_ref,
              *out_refs):
    pst_refs = out_refs[:NCH]
    dinv_ref, rdinv_ref = out_refs[NCH], out_refs[NCH + 1]
    h = jax.nn.relu(
        jnp.dot(w0t_ref[...], xt_ref[...], preferred_element_type=jnp.float32)
        + b0_ref[...])
    predst = (jnp.dot(w1t_ref[...], h, preferred_element_type=jnp.float32)
              + b1_ref[...])
    deg = degp_ref[0:1, :] + degp_ref[1:2, :] + 1.0
    dinv = 1.0 / jnp.sqrt(deg)
    dinv_ref[...] = dinv
    rdinv_ref[...] = jnp.sqrt(deg)
    pst = predst * dinv
    for k in range(NCH):
        pst_refs[k][...] = pst[k * CH:(k + 1) * CH, :]


def _mlp_prep(xt, W0t, b0, W1t, b1, degp):
    return pl.pallas_call(
        _mlp_body,
        grid=(GRID,),
        in_specs=[
            pl.BlockSpec((F, RB), lambda i: (0, i)),
            pl.BlockSpec((H, F), lambda i: (0, 0)),
            pl.BlockSpec((H, 1), lambda i: (0, 0)),
            pl.BlockSpec((C, H), lambda i: (0, 0)),
            pl.BlockSpec((C, 1), lambda i: (0, 0)),
            pl.BlockSpec((2, RB), lambda i: (0, i)),
        ],
        out_specs=[pl.BlockSpec((CH, RB), lambda i: (0, i))] * NCH + [
            pl.BlockSpec((1, RB), lambda i: (0, i)),
            pl.BlockSpec((1, RB), lambda i: (0, i)),
        ],
        out_shape=[jax.ShapeDtypeStruct((CH, NACC), jnp.float32)] * NCH + [
            jax.ShapeDtypeStruct((1, NACC), jnp.float32),
            jax.ShapeDtypeStruct((1, NACC), jnp.float32),
        ],
    )(xt, W0t, b0.reshape(H, 1), W1t, b1.reshape(C, 1), degp)


def _halt_body(*refs):
    rawt_refs = refs[:NCH]
    pst_refs = refs[NCH:2 * NCH]
    (dinv_ref, rdinv_ref, steps_ref, sumh_ref, cont_ref, xacct_ref,
     wh_ref, bh_ref) = refs[2 * NCH:2 * NCH + 8]
    out = refs[2 * NCH + 8:]
    opst_refs = out[:NCH]
    (osteps_ref, osumh_ref, ocont_ref, oxacct_ref) = out[NCH:]
    raw = jnp.concatenate([r[...] for r in rawt_refs], axis=0)
    ps = jnp.concatenate([p[...] for p in pst_refs], axis=0)
    dinv = dinv_ref[...]
    prop = dinv * (raw + ps)
    old_prop = rdinv_ref[...] * ps
    hh = jax.nn.sigmoid(
        jnp.sum(prop * wh_ref[...], axis=0, keepdims=True) + bh_ref[0, 0])
    steps = steps_ref[...]
    sum_h = sumh_ref[...]
    cont = cont_ref[...]
    prob = jnp.where((sum_h + hh < 0.99) & (cont > 0.0), 1.0, 0.0)
    steps = steps + prob
    sum_h = sum_h + prob * hh
    condition = prob * jnp.where(steps < float(NITER), 1.0, 0.0)
    p = jnp.where(condition > 0.0, sum_h, 1.0 - sum_h)
    oxacct_ref[...] = xacct_ref[...] + (
        prop * p + old_prop * (1.0 - p)) * cont
    nps = dinv * prop
    for k in range(NCH):
        opst_refs[k][...] = nps[k * CH:(k + 1) * CH, :]
    osteps_ref[...] = steps
    osumh_ref[...] = sum_h
    ocont_ref[...] = cont * prob


def _halt(rawts, psts, dinv, rdinv, steps, sum_h, cont, xacct, Wh, bh):
    cch = lambda: pl.BlockSpec((CH, RB), lambda i: (0, i))
    c1 = lambda: pl.BlockSpec((1, RB), lambda i: (0, i))
    return pl.pallas_call(
        _halt_body,
        grid=(GRID,),
        in_specs=[cch() for _ in range(2 * NCH)]
                 + [c1(), c1(), c1(), c1(), c1(),
                    pl.BlockSpec((C, RB), lambda i: (0, i)),
                    pl.BlockSpec((C, 1), lambda i: (0, 0)),
                    pl.BlockSpec((1, 1), lambda i: (0, 0))],
        out_specs=[cch() for _ in range(NCH)]
                  + [c1(), c1(), c1(),
                     pl.BlockSpec((C, RB), lambda i: (0, i))],
        out_shape=[jax.ShapeDtypeStruct((CH, NACC), jnp.float32)] * NCH + [
            jax.ShapeDtypeStruct((1, NACC), jnp.float32),
            jax.ShapeDtypeStruct((1, NACC), jnp.float32),
            jax.ShapeDtypeStruct((1, NACC), jnp.float32),
            jax.ShapeDtypeStruct((C, NACC), jnp.float32),
        ],
    )(*rawts, *psts, dinv, rdinv, steps, sum_h, cont, xacct, Wh, bh)


def _final_body(xacct_ref, steps_ref, sumh_ref, out_ref, osteps_ref, orem_ref):
    steps = steps_ref[...]
    o = xacct_ref[...] / steps
    m = jnp.max(o, axis=0, keepdims=True)
    z = o - m
    lse = jnp.log(jnp.sum(jnp.exp(z), axis=0, keepdims=True))
    out_ref[...] = z - lse
    osteps_ref[...] = steps
    orem_ref[...] = 1.0 - sumh_ref[...]


def _final(xacct, steps, sum_h):
    c1 = lambda: pl.BlockSpec((1, RB), lambda i: (0, i))
    cc = lambda: pl.BlockSpec((C, RB), lambda i: (0, i))
    return pl.pallas_call(
        _final_body,
        grid=(GRID,),
        in_specs=[cc(), c1(), c1()],
        out_specs=[cc(), c1(), c1()],
        out_shape=[
            jax.ShapeDtypeStruct((C, NACC), jnp.float32),
            jax.ShapeDtypeStruct((1, NACC), jnp.float32),
            jax.ShapeDtypeStruct((1, NACC), jnp.float32),
        ],
    )(xacct, steps, sum_h)


def kernel(x, edge_index, W0, b0, W1, b1, Wh, bh):
    src = edge_index[0]
    dst = edge_index[1]
    padidx = (N + (jnp.arange(E_PAD - E, dtype=jnp.int32) % (NACC - N)))
    srcg = jnp.concatenate([src, padidx]).reshape(GROUPS, 128)
    dstg = jnp.concatenate([dst, padidx]).reshape(GROUPS, 128)
    zeros1 = jnp.zeros((NACC,), jnp.float32)
    zeros2 = jnp.zeros((CO, CH), jnp.float32)
    xt = jnp.pad(x.T, ((0, 0), (0, NACC - N)))

    degp = _deg_sc(srcg, zeros1)
    *psts, dinv, rdinv = _mlp_prep(xt, W0.T, b0, W1.T, b1, degp)

    bh2 = bh.reshape(1, 1)
    steps = jnp.ones((1, NACC), jnp.float32)
    sum_h = jnp.zeros((1, NACC), jnp.float32)
    cont = jnp.ones((1, NACC), jnp.float32)
    xacct = jnp.zeros((C, NACC), jnp.float32)

    for _ in range(NITER):
        raws = [_prop_sc(pst.T, srcg, dstg, zeros2) for pst in psts]
        rawts = [(r[0] + r[1]).T for r in raws]
        *psts_new, steps, sum_h, cont, xacct = _halt(
            rawts, psts, dinv, rdinv, steps, sum_h, cont, xacct, Wh, bh2)
        psts = psts_new

    logitst, steps_o, rem = _final(xacct, steps, sum_h)
    return (logitst[:, :N].T, steps_o[0, :N], rem[0, :N])


# speedup vs baseline: 10.6021x; 1.0560x over previous
import functools
import math

import jax
import jax.numpy as jnp
from jax import lax
from jax.experimental import pallas as pl
from jax.experimental.pallas import tpu as pltpu
from jax.experimental.pallas import tpu_sc as plsc

N = 100000
E = 1600000
F = 128
H = 64
C = 40
NITER = 10
CH = 8
NCH = C // CH

NACC = 100352
STRIPE = NACC // 16
CO = STRIPE // 4

E_PAD = 1601536
GROUPS = E_PAD // 128
GM = 17
LMS = GM * 128
GPTD = GROUPS // 32
NBD = GPTD // GM

RB = STRIPE
GRID = NACC // RB

_mesh = plsc.VectorSubcoreMesh(core_axis_name="c", subcore_axis_name="s")


def _deg_body(srcg, zeros1, degp, onesb, idxb, acc1, obuf1, dsem):
    cid = lax.axis_index("c")
    sid = lax.axis_index("s")
    for k in range(128 // 16):
        onesb[pl.ds(k * 16, 16)] = jnp.full((16,), 1.0, jnp.float32)
    @pl.loop(0, 2)
    def _z(k):
        off = sid * STRIPE + k * (STRIPE // 2)
        pltpu.sync_copy(zeros1.at[pl.ds(off, STRIPE // 2)], obuf1)
        pltpu.sync_copy(obuf1, acc1.at[pl.ds(off, STRIPE // 2)])

    plsc.subcore_barrier()
    tb = cid * (GROUPS // 2) + sid * GPTD

    @pl.loop(0, NBD)
    def _blk(b):
        gb = tb + b * GM
        pltpu.sync_copy(srcg.at[pl.ds(gb, GM)], idxb)
        for g in range(GM):
            pltpu.async_copy(onesb, acc1.at[idxb.at[g]], dsem, add=True)
        for g in range(GM):
            pltpu.make_async_copy(onesb, acc1.at[idxb.at[g]], dsem).wait()

    plsc.subcore_barrier()

    @pl.loop(0, 2)
    def _co(k):
        off = sid * STRIPE + k * (STRIPE // 2)
        pltpu.sync_copy(acc1.at[pl.ds(off, STRIPE // 2)], obuf1)
        pltpu.sync_copy(obuf1, degp.at[cid, pl.ds(off, STRIPE // 2)])


@functools.partial(
    pl.kernel,
    out_type=jax.ShapeDtypeStruct((2, NACC), jnp.float32),
    mesh=_mesh,
    compiler_params=pltpu.CompilerParams(use_tc_tiling_on_sc=False),
    scratch_types=[
        pltpu.VMEM((128,), jnp.float32),
        pltpu.VMEM((GM, 128), jnp.int32),
        pltpu.VMEM_SHARED((NACC,), jnp.float32),
        pltpu.VMEM((STRIPE // 2,), jnp.float32),
        pltpu.SemaphoreType.DMA,
    ],
)
def _deg_sc(srcg, zeros1, degp, onesb, idxb, acc1, obuf1, dsem):
    _deg_body(srcg, zeros1, degp, onesb, idxb, acc1, obuf1, dsem)


@functools.partial(
    pl.kernel,
    out_type=jax.ShapeDtypeStruct((2, NCH * NACC, CH), jnp.float32),
    mesh=_mesh,
    compiler_params=pltpu.CompilerParams(use_tc_tiling_on_sc=False),
    scratch_types=[
        pltpu.VMEM((2, GM, 128), jnp.int32),
        pltpu.VMEM((2, GM, 128), jnp.int32),
        pltpu.VMEM((2, LMS, CH), jnp.float32),
        pltpu.VMEM((CO, CH), jnp.float32),
        pltpu.VMEM_SHARED((NACC, CH), jnp.float32),
        pltpu.SemaphoreType.DMA,
        pltpu.SemaphoreType.DMA,
    ],
)
def _prop_sc(ps_all, srcg_all, dstg, zeros2, rawp,
             srcb, dstb, rows, zbuf, acc, gsem, ssem):
    cid = lax.axis_index("c")
    sid = lax.axis_index("s")
    pltpu.sync_copy(zeros2, zbuf)

    @pl.loop(0, 4)
    def _z(j):
        off = sid * STRIPE + j * CO
        pltpu.sync_copy(zbuf, acc.at[pl.ds(off, CO)])

    plsc.subcore_barrier()
    tbd = cid * (GROUPS // 2) + sid * GPTD

    for k in range(NCH):
        tbs = k * GROUPS + tbd
        pltpu.sync_copy(srcg_all.at[pl.ds(tbs, GM)], srcb.at[0])
        pltpu.sync_copy(dstg.at[pl.ds(tbd, GM)], dstb.at[0])
        for g in range(GM):
            pltpu.async_copy(ps_all.at[srcb.at[0, g]],
                             rows.at[0, pl.ds(g * 128, 128)], gsem)

        @pl.loop(0, NBD)
        def _blk(b):
            p = lax.rem(b, 2)
            q = 1 - p

            @pl.when(b < NBD - 1)
            def _():
                pltpu.sync_copy(srcg_all.at[pl.ds(tbs + (b + 1) * GM, GM)],
                                srcb.at[q])
                pltpu.sync_copy(dstg.at[pl.ds(tbd + (b + 1) * GM, GM)],
                                dstb.at[q])

            for g in range(GM):
                pltpu.make_async_copy(ps_all.at[srcb.at[p, g]],
                                      rows.at[p, pl.ds(g * 128, 128)],
                                      gsem).wait()

            @pl.when(b < NBD - 1)
            def _():
                for g in range(GM):
                    pltpu.async_copy(ps_all.at[srcb.at[q, g]],
                                     rows.at[q, pl.ds(g * 128, 128)], gsem)

            for g in range(GM):
                pltpu.async_copy(rows.at[p, pl.ds(g * 128, 128)],
                                 acc.at[dstb.at[p, g]], ssem, add=True)
            for g in range(GM):
                pltpu.make_async_copy(rows.at[p, pl.ds(g * 128, 128)],
                                      acc.at[dstb.at[p, g]], ssem).wait()

        plsc.subcore_barrier()

        @pl.loop(0, 4)
        def _co(j):
            off = sid * STRIPE + j * CO
            pltpu.sync_copy(acc.at[pl.ds(off, CO)], rows.at[0, pl.ds(0, CO)])
            pltpu.sync_copy(rows.at[0, pl.ds(0, CO)],
                            rawp.at[cid, pl.ds(k * NACC + off, CO)])

        if k < NCH - 1:
            @pl.loop(0, 4)
            def _z2(j):
                off = sid * STRIPE + j * CO
                pltpu.sync_copy(zbuf, acc.at[pl.ds(off, CO)])

            plsc.subcore_barrier()


def _mlp_body(xt_ref, w0t_ref, b0_ref, w1t_ref, b1_ref, degp_ref,
              pst_ref, dinv_ref, rdinv_ref):
    h = jax.nn.relu(
        jnp.dot(w0t_ref[...], xt_ref[...], preferred_element_type=jnp.float32)
        + b0_ref[...])
    predst = (jnp.dot(w1t_ref[...], h, preferred_element_type=jnp.float32)
              + b1_ref[...])
    deg = degp_ref[0:1, :] + degp_ref[1:2, :] + 1.0
    dinv = 1.0 / jnp.sqrt(deg)
    dinv_ref[...] = dinv
    rdinv_ref[...] = jnp.sqrt(deg)
    pst_ref[...] = predst * dinv


def _mlp_prep(xt, W0t, b0, W1t, b1, degp):
    return pl.pallas_call(
        _mlp_body,
        grid=(GRID,),
        in_specs=[
            pl.BlockSpec((F, RB), lambda i: (0, i)),
            pl.BlockSpec((H, F), lambda i: (0, 0)),
            pl.BlockSpec((H, 1), lambda i: (0, 0)),
            pl.BlockSpec((C, H), lambda i: (0, 0)),
            pl.BlockSpec((C, 1), lambda i: (0, 0)),
            pl.BlockSpec((2, RB), lambda i: (0, i)),
        ],
        out_specs=[pl.BlockSpec((C, RB), lambda i: (0, i)),
                   pl.BlockSpec((1, RB), lambda i: (0, i)),
                   pl.BlockSpec((1, RB), lambda i: (0, i))],
        out_shape=[jax.ShapeDtypeStruct((C, NACC), jnp.float32),
                   jax.ShapeDtypeStruct((1, NACC), jnp.float32),
                   jax.ShapeDtypeStruct((1, NACC), jnp.float32)],
    )(xt, W0t, b0.reshape(H, 1), W1t, b1.reshape(C, 1), degp)


def _halt_body(rawt_ref, pst_ref, dinv_ref, rdinv_ref, steps_ref,
               sumh_ref, cont_ref, xacct_ref, wh_ref, bh_ref,
               opst_ref, osteps_ref, osumh_ref, ocont_ref, oxacct_ref):
    raw = rawt_ref[...]
    ps = pst_ref[...]
    dinv = dinv_ref[...]
    prop = dinv * (raw + ps)
    old_prop = rdinv_ref[...] * ps
    hh = jax.nn.sigmoid(
        jnp.sum(prop * wh_ref[...], axis=0, keepdims=True) + bh_ref[0, 0])
    steps = steps_ref[...]
    sum_h = sumh_ref[...]
    cont = cont_ref[...]
    prob = jnp.where((sum_h + hh < 0.99) & (cont > 0.0), 1.0, 0.0)
    steps = steps + prob
    sum_h = sum_h + prob * hh
    condition = prob * jnp.where(steps < float(NITER), 1.0, 0.0)
    p = jnp.where(condition > 0.0, sum_h, 1.0 - sum_h)
    oxacct_ref[...] = xacct_ref[...] + (
        prop * p + old_prop * (1.0 - p)) * cont
    opst_ref[...] = dinv * prop
    osteps_ref[...] = steps
    osumh_ref[...] = sum_h
    ocont_ref[...] = cont * prob


def _halt(rawt, pst, dinv, rdinv, steps, sum_h, cont, xacct, Wh, bh):
    cc = lambda: pl.BlockSpec((C, RB), lambda i: (0, i))
    c1 = lambda: pl.BlockSpec((1, RB), lambda i: (0, i))
    return pl.pallas_call(
        _halt_body,
        grid=(GRID,),
        in_specs=[cc(), cc(), c1(), c1(), c1(), c1(), c1(), cc(),
                  pl.BlockSpec((C, 1), lambda i: (0, 0)),
                  pl.BlockSpec((1, 1), lambda i: (0, 0))],
        out_specs=[cc(), c1(), c1(), c1(), cc()],
        out_shape=[
            jax.ShapeDtypeStruct((C, NACC), jnp.float32),
            jax.ShapeDtypeStruct((1, NACC), jnp.float32),
            jax.ShapeDtypeStruct((1, NACC), jnp.float32),
            jax.ShapeDtypeStruct((1, NACC), jnp.float32),
            jax.ShapeDtypeStruct((C, NACC), jnp.float32),
        ],
    )(rawt, pst, dinv, rdinv, steps, sum_h, cont, xacct, Wh, bh)


def _final_body(xacct_ref, steps_ref, sumh_ref, out_ref, osteps_ref, orem_ref):
    steps = steps_ref[...]
    o = xacct_ref[...] / steps
    m = jnp.max(o, axis=0, keepdims=True)
    z = o - m
    lse = jnp.log(jnp.sum(jnp.exp(z), axis=0, keepdims=True))
    out_ref[...] = z - lse
    osteps_ref[...] = steps
    orem_ref[...] = 1.0 - sumh_ref[...]


def _final(xacct, steps, sum_h):
    c1 = lambda: pl.BlockSpec((1, RB), lambda i: (0, i))
    cc = lambda: pl.BlockSpec((C, RB), lambda i: (0, i))
    return pl.pallas_call(
        _final_body,
        grid=(GRID,),
        in_specs=[cc(), c1(), c1()],
        out_specs=[cc(), c1(), c1()],
        out_shape=[
            jax.ShapeDtypeStruct((C, NACC), jnp.float32),
            jax.ShapeDtypeStruct((1, NACC), jnp.float32),
            jax.ShapeDtypeStruct((1, NACC), jnp.float32),
        ],
    )(xacct, steps, sum_h)


def kernel(x, edge_index, W0, b0, W1, b1, Wh, bh):
    src = edge_index[0]
    dst = edge_index[1]
    padidx = (N + (jnp.arange(E_PAD - E, dtype=jnp.int32) % (NACC - N)))
    srcg1 = jnp.concatenate([src, padidx])
    srcg_all = jnp.concatenate(
        [srcg1 + k * NACC for k in range(NCH)]).reshape(NCH * GROUPS, 128)
    srcg = srcg1.reshape(GROUPS, 128)
    dstg = jnp.concatenate([dst, padidx]).reshape(GROUPS, 128)
    zeros1 = jnp.zeros((NACC,), jnp.float32)
    zeros2 = jnp.zeros((CO, CH), jnp.float32)
    xt = jnp.pad(x.T, ((0, 0), (0, NACC - N)))

    degp = _deg_sc(srcg, zeros1)
    pst, dinv, rdinv = _mlp_prep(xt, W0.T, b0, W1.T, b1, degp)

    bh2 = bh.reshape(1, 1)
    steps = jnp.ones((1, NACC), jnp.float32)
    sum_h = jnp.zeros((1, NACC), jnp.float32)
    cont = jnp.ones((1, NACC), jnp.float32)
    xacct = jnp.zeros((C, NACC), jnp.float32)

    for _ in range(NITER):
        ps_all = pst.reshape(NCH, CH, NACC).transpose(0, 2, 1).reshape(
            NCH * NACC, CH)
        rawp = _prop_sc(ps_all, srcg_all, dstg, zeros2)
        rawt = (rawp[0] + rawp[1]).reshape(NCH, NACC, CH).transpose(
            0, 2, 1).reshape(C, NACC)
        pst, steps, sum_h, cont, xacct = _halt(
            rawt, pst, dinv, rdinv, steps, sum_h, cont, xacct, Wh, bh2)

    logitst, steps_o, rem = _final(xacct, steps, sum_h)
    return (logitst[:, :N].T, steps_o[0, :N], rem[0, :N])
